# baseline (speedup 1.0000x reference)
import functools

import jax
import jax.numpy as jnp
from jax.experimental import pallas as pl
from jax.experimental.pallas import tpu as pltpu

_K = 32
_BR1 = 512
_BH1 = 512
_BR2 = 2048
_BH2 = 512


def _key_to_f32(k):
    neg = jax.lax.shift_right_arithmetic(k, 31)
    return jax.lax.bitcast_convert_type(k ^ (neg & jnp.int32(0x7FFFFFFF)),
                                        jnp.float32)


def _enc_kernel(nbj, x_ref, we_ref, be_ref, h_ref, thr_ref, h_scr):
    j = pl.program_id(1)
    br = x_ref.shape[0]

    h_tile = jax.lax.dot_general(
        x_ref[...], we_ref[...], (((1,), (1,)), ((), ())),
        preferred_element_type=jnp.float32) + be_ref[...]
    h_ref[...] = h_tile
    h_scr[j] = h_tile

    @pl.when(j == nbj - 1)
    def _threshold():
        def count_ge(cand_f):
            def body(t, acc):
                chunk = h_scr[t]
                return acc + jnp.sum((chunk >= cand_f).astype(jnp.int32),
                                     axis=1, keepdims=True)
            return jax.lax.fori_loop(0, nbj, body,
                                     jnp.zeros((br, 1), jnp.int32))

        cnt0 = count_ge(jnp.zeros((br, 1), jnp.float32))
        t = jnp.where(cnt0 >= _K, jnp.int32(0), jnp.int32(-2147483648))

        def bit_body(i, t):
            bit = jax.lax.shift_left(jnp.int32(1), 30 - i)
            cand = t | bit
            cnt = count_ge(_key_to_f32(cand))
            return jnp.where(cnt >= _K, cand, t)

        t = jax.lax.fori_loop(0, 31, bit_body, t)
        thr_ref[...] = _key_to_f32(t)


def _dec_kernel(h_ref, thr_ref, wd_ref, bd_ref, recon_ref, hs_ref):
    j = pl.program_id(1)
    h_tile = h_ref[...]
    hs = jnp.where(h_tile >= thr_ref[...], h_tile, jnp.float32(0.0))
    hs_ref[...] = hs
    part = jax.lax.dot_general(
        hs, wd_ref[...], (((1,), (1,)), ((), ())),
        preferred_element_type=jnp.float32)

    @pl.when(j == 0)
    def _():
        recon_ref[...] = part + bd_ref[...]

    @pl.when(j > 0)
    def _():
        recon_ref[...] += part


def kernel(x, W_e, b_e, W_d, b_d):
    B, D = x.shape
    H = W_e.shape[0]
    br1 = _BR1 if B % _BR1 == 0 else B
    bh1 = _BH1 if H % _BH1 == 0 else H
    br2 = _BR2 if B % _BR2 == 0 else B
    bh2 = _BH2 if H % _BH2 == 0 else H
    nbj1 = H // bh1

    be2 = b_e.reshape(1, H)
    bd2 = b_d.reshape(1, D)

    h, thr = pl.pallas_call(
        functools.partial(_enc_kernel, nbj1),
        grid=(B // br1, nbj1),
        in_specs=[
            pl.BlockSpec((br1, D), lambda i, j: (i, 0)),
            pl.BlockSpec((bh1, D), lambda i, j: (j, 0)),
            pl.BlockSpec((1, bh1), lambda i, j: (0, j)),
        ],
        out_specs=[
            pl.BlockSpec((br1, bh1), lambda i, j: (i, j)),
            pl.BlockSpec((br1, 1), lambda i, j: (i, 0)),
        ],
        out_shape=[
            jax.ShapeDtypeStruct((B, H), jnp.float32),
            jax.ShapeDtypeStruct((B, 1), jnp.float32),
        ],
        scratch_shapes=[
            pltpu.VMEM((nbj1, br1, bh1), jnp.float32),
        ],
    )(x, W_e, be2)

    recon, hs = pl.pallas_call(
        _dec_kernel,
        grid=(B // br2, H // bh2),
        in_specs=[
            pl.BlockSpec((br2, bh2), lambda i, j: (i, j)),
            pl.BlockSpec((br2, 1), lambda i, j: (i, 0)),
            pl.BlockSpec((D, bh2), lambda i, j: (0, j)),
            pl.BlockSpec((1, D), lambda i, j: (0, 0)),
        ],
        out_specs=[
            pl.BlockSpec((br2, D), lambda i, j: (i, 0)),
            pl.BlockSpec((br2, bh2), lambda i, j: (i, j)),
        ],
        out_shape=[
            jax.ShapeDtypeStruct((B, D), jnp.float32),
            jax.ShapeDtypeStruct((B, H), jnp.float32),
        ],
    )(h, thr, W_d, bd2)
    return (recon, hs, h)
